# baseline (speedup 1.0000x reference)
import jax
import jax.numpy as jnp
from jax.experimental import pallas as pl
from jax.experimental.pallas import tpu as pltpu

EPS = 1e-05
_BLOCK_ROWS = 4096


def _pdf_block(w_ref, o_ref):
    w = w_ref[...]
    s = jnp.sum(w, axis=1, keepdims=True)
    pad = jnp.maximum(EPS - s, 0.0)
    o_ref[...] = (w + pad * (1.0 / w.shape[1])) / (s + pad)


def kernel(weights, stratified):
    n, d = weights.shape
    return pl.pallas_call(
        _pdf_block,
        grid=(n // _BLOCK_ROWS,),
        in_specs=[pl.BlockSpec((_BLOCK_ROWS, d), lambda i: (i, 0))],
        out_specs=pl.BlockSpec((_BLOCK_ROWS, d), lambda i: (i, 0)),
        out_shape=jax.ShapeDtypeStruct((n, d), weights.dtype),
        compiler_params=pltpu.CompilerParams(
            dimension_semantics=("parallel",),
        ),
    )(weights)
